# baseline (speedup 1.0000x reference)
import jax
import jax.numpy as jnp
from jax import lax
from jax.experimental import pallas as pl
from jax.experimental.pallas import tpu as pltpu
from jax.experimental.pallas import tpu_sc as plsc

N = 10000
E = 320000
D = 128
H = 64
O = 2

NC = 2
NS = 16
NW = NC * NS
EPW = E // NW
C = 80
NCH = EPW // C
NP = 10240
RPT = NP // NS
ZB = 128

_f32 = jnp.float32


def _seg_kernel(width):
    mesh = plsc.VectorSubcoreMesh(core_axis_name="c", subcore_axis_name="s")
    gpr = width // 16

    def body(p_hbm, src_hbm, dst_hbm, out_hbm, idx_s, idx_d, rows, zbuf, table,
             isem, gsem, ssem):
        c = lax.axis_index("c")
        s = lax.axis_index("s")
        wid = c * NS + s

        pltpu.async_copy(src_hbm.at[wid], idx_s, isem)
        pltpu.async_copy(dst_hbm.at[wid], idx_d, isem)

        def zs(t, carry):
            zbuf[t // gpr, pl.ds((t % gpr) * 16, 16)] = jnp.zeros((16,), _f32)
            return carry

        lax.fori_loop(0, ZB * gpr, zs, 0)
        base_r = s * RPT
        for z in range(RPT // ZB):
            pltpu.sync_copy(zbuf, table.at[pl.ds(base_r + z * ZB, ZB)])
        pltpu.make_async_copy(src_hbm.at[wid], idx_s, isem).wait()
        pltpu.make_async_copy(dst_hbm.at[wid], idx_d, isem).wait()
        plsc.subcore_barrier()

        pltpu.async_copy(p_hbm.at[idx_s.at[0]], rows.at[0], gsem)

        def pair(m, carry):
            for b in range(2):
                j = 2 * m + b

                @pl.when(j < NCH)
                def _():
                    pltpu.make_async_copy(
                        p_hbm.at[pl.ds(0, C)], rows.at[b], gsem).wait()

                    @pl.when(j >= 1)
                    def _():
                        pltpu.make_async_copy(
                            rows.at[1 - b], table.at[pl.ds(0, C)], ssem).wait()

                    @pl.when(j + 1 < NCH)
                    def _():
                        pltpu.async_copy(
                            p_hbm.at[idx_s.at[j + 1]], rows.at[1 - b], gsem)

                    pltpu.async_copy(
                        rows.at[b], table.at[idx_d.at[j]], ssem, add=True)
            return carry

        lax.fori_loop(0, (NCH + 1) // 2, pair, 0)
        pltpu.make_async_copy(
            rows.at[(NCH - 1) % 2], table.at[pl.ds(0, C)], ssem).wait()
        plsc.subcore_barrier()
        pltpu.sync_copy(table.at[pl.ds(base_r, RPT)],
                        out_hbm.at[c, pl.ds(base_r, RPT)])

    return pl.kernel(
        body,
        out_type=jax.ShapeDtypeStruct((NC, NP, width), _f32),
        mesh=mesh,
        compiler_params=pltpu.CompilerParams(use_tc_tiling_on_sc=False),
        scratch_types=[
            pltpu.VMEM((NCH, C), jnp.int32),
            pltpu.VMEM((NCH, C), jnp.int32),
            pltpu.VMEM((2, C, width), _f32),
            pltpu.VMEM((ZB, width), _f32),
            pltpu.VMEM_SHARED((NP, width), _f32),
            pltpu.SemaphoreType.DMA,
            pltpu.SemaphoreType.DMA,
            pltpu.SemaphoreType.DMA,
        ],
    )


def _edge_kernel():
    mesh = plsc.VectorSubcoreMesh(core_axis_name="c", subcore_axis_name="s")

    def body(a_hbm, b_hbm, src_hbm, dst_hbm, wm2t_hbm, bm2_hbm, out_hbm,
             idx_s, idx_d, ra, rb, lg, wv, bv, gsem, wsem):
        c = lax.axis_index("c")
        s = lax.axis_index("s")
        wid = c * NS + s
        pltpu.sync_copy(wm2t_hbm, wv)
        pltpu.sync_copy(bm2_hbm, bv)
        pltpu.sync_copy(src_hbm.at[wid], idx_s)
        pltpu.sync_copy(dst_hbm.at[wid], idx_d)

        w0 = [wv[0, pl.ds(k * 16, 16)] for k in range(H // 16)]
        w1 = [wv[1, pl.ds(k * 16, 16)] for k in range(H // 16)]
        bvec = bv[pl.ds(0, 16)]
        b0 = bvec[0]
        b1 = bvec[1]
        io = lax.iota(jnp.int32, 16)
        io_lt2 = io < 2
        io_eq1 = io == 1

        pltpu.async_copy(a_hbm.at[idx_s.at[0]], ra.at[0], gsem)
        pltpu.async_copy(b_hbm.at[idx_d.at[0]], rb.at[0], gsem)

        def pair(m, carry):
            for b in range(2):
                j = 2 * m + b

                @pl.when(j < NCH)
                def _():
                    pltpu.make_async_copy(
                        a_hbm.at[pl.ds(0, C)], ra.at[b], gsem).wait()
                    pltpu.make_async_copy(
                        b_hbm.at[pl.ds(0, C)], rb.at[b], gsem).wait()

                    @pl.when(j >= 1)
                    def _():
                        pltpu.make_async_copy(
                            lg.at[1 - b], out_hbm.at[pl.ds(0, C * O)],
                            wsem).wait()

                    @pl.when(j + 1 < NCH)
                    def _():
                        pltpu.async_copy(
                            a_hbm.at[idx_s.at[j + 1]], ra.at[1 - b], gsem)
                        pltpu.async_copy(
                            b_hbm.at[idx_d.at[j + 1]], rb.at[1 - b], gsem)

                    rab = ra.at[b]
                    rbb = rb.at[b]
                    lgb = lg.at[b]

                    def vop(i, cc):
                        g = [jnp.maximum(rab[i, pl.ds(k * 16, 16)]
                                         + rbb[i, pl.ds(k * 16, 16)], 0.0)
                             for k in range(H // 16)]
                        s0 = g[0] * w0[0]
                        s1 = g[0] * w1[0]
                        for k in range(1, H // 16):
                            s0 = s0 + g[k] * w0[k]
                            s1 = s1 + g[k] * w1[k]
                        v = jnp.where(io_eq1, jnp.sum(s1) + b1,
                                      jnp.sum(s0) + b0)
                        plsc.store_scatter(lgb, [io + 2 * i], v, mask=io_lt2)
                        return cc

                    lax.fori_loop(0, C, vop, 0)
                    pltpu.async_copy(
                        lgb, out_hbm.at[pl.ds((wid * EPW + j * C) * O, C * O)],
                        wsem)
            return carry

        lax.fori_loop(0, (NCH + 1) // 2, pair, 0)
        pltpu.make_async_copy(
            lg.at[(NCH - 1) % 2], out_hbm.at[pl.ds(0, C * O)], wsem).wait()

    return pl.kernel(
        body,
        out_type=jax.ShapeDtypeStruct((E * O,), _f32),
        mesh=mesh,
        compiler_params=pltpu.CompilerParams(
            use_tc_tiling_on_sc=False, needs_layout_passes=False),
        scratch_types=[
            pltpu.VMEM((NCH, C), jnp.int32),
            pltpu.VMEM((NCH, C), jnp.int32),
            pltpu.VMEM((2, C, H), _f32),
            pltpu.VMEM((2, C, H), _f32),
            pltpu.VMEM((2, C * O), _f32),
            pltpu.VMEM((O, H), _f32),
            pltpu.VMEM((16,), _f32),
            pltpu.SemaphoreType.DMA,
            pltpu.SemaphoreType.DMA,
        ],
    )


_seg80 = _seg_kernel(H + 16)
_seg64 = _seg_kernel(H)
_edge = _edge_kernel()

RB = 2000


def _t1_body(x_ref, w1l_ref, w1r_ref, paug_ref, r1_ref):
    xb = x_ref[...]
    p1 = jnp.dot(xb, w1l_ref[...], preferred_element_type=_f32)
    paug_ref[...] = jnp.concatenate(
        [p1, jnp.ones((xb.shape[0], 16), _f32)], axis=1)
    r1_ref[...] = jnp.dot(xb, w1r_ref[...], preferred_element_type=_f32)


_t1 = pl.pallas_call(
    _t1_body,
    grid=(N // RB,),
    in_specs=[
        pl.BlockSpec((RB, D), lambda i: (i, 0)),
        pl.BlockSpec((D, H), lambda i: (0, 0)),
        pl.BlockSpec((D, H), lambda i: (0, 0)),
    ],
    out_specs=[
        pl.BlockSpec((RB, H + 16), lambda i: (i, 0)),
        pl.BlockSpec((RB, H), lambda i: (i, 0)),
    ],
    out_shape=[
        jax.ShapeDtypeStruct((N, H + 16), _f32),
        jax.ShapeDtypeStruct((N, H), _f32),
    ],
)


def _t2_body(tab_ref, r1_ref, b1l_ref, w2l_ref, w2r_ref, p2_ref, r2_ref, inv_ref):
    tab = tab_ref[...]
    agg = tab[0, :, :H] + tab[1, :, :H]
    cnt = tab[0, :, H:H + 1] + tab[1, :, H:H + 1]
    inv = 1.0 / jnp.maximum(cnt, 1.0)
    h1 = jnp.maximum(agg * inv + b1l_ref[...][None, :] + r1_ref[...], 0.0)
    p2_ref[...] = jnp.dot(h1, w2l_ref[...], preferred_element_type=_f32)
    r2_ref[...] = jnp.dot(h1, w2r_ref[...], preferred_element_type=_f32)
    inv_ref[...] = jnp.broadcast_to(inv, (inv.shape[0], 8))


_t2 = pl.pallas_call(
    _t2_body,
    grid=(N // RB,),
    in_specs=[
        pl.BlockSpec((NC, RB, H + 16), lambda i: (0, i, 0)),
        pl.BlockSpec((RB, H), lambda i: (i, 0)),
        pl.BlockSpec((H,), lambda i: (0,)),
        pl.BlockSpec((H, H), lambda i: (0, 0)),
        pl.BlockSpec((H, H), lambda i: (0, 0)),
    ],
    out_specs=[
        pl.BlockSpec((RB, H), lambda i: (i, 0)),
        pl.BlockSpec((RB, H), lambda i: (i, 0)),
        pl.BlockSpec((RB, 8), lambda i: (i, 0)),
    ],
    out_shape=[
        jax.ShapeDtypeStruct((N, H), _f32),
        jax.ShapeDtypeStruct((N, H), _f32),
        jax.ShapeDtypeStruct((N, 8), _f32),
    ],
)


def _t3_body(tab_ref, r2_ref, inv_ref, b2l_ref, wm1_ref, bm1_ref, a_ref, b_ref):
    tab = tab_ref[...]
    agg = tab[0] + tab[1]
    inv = inv_ref[...][:, :1]
    h2 = jnp.maximum(agg * inv + b2l_ref[...][None, :] + r2_ref[...], 0.0)
    wm1 = wm1_ref[...]
    a_ref[...] = jnp.dot(h2, wm1[:H], preferred_element_type=_f32) \
        + bm1_ref[...][None, :]
    b_ref[...] = jnp.dot(h2, wm1[H:], preferred_element_type=_f32)


_t3 = pl.pallas_call(
    _t3_body,
    grid=(N // RB,),
    in_specs=[
        pl.BlockSpec((NC, RB, H), lambda i: (0, i, 0)),
        pl.BlockSpec((RB, H), lambda i: (i, 0)),
        pl.BlockSpec((RB, 8), lambda i: (i, 0)),
        pl.BlockSpec((H,), lambda i: (0,)),
        pl.BlockSpec((2 * H, H), lambda i: (0, 0)),
        pl.BlockSpec((H,), lambda i: (0,)),
    ],
    out_specs=[
        pl.BlockSpec((RB, H), lambda i: (i, 0)),
        pl.BlockSpec((RB, H), lambda i: (i, 0)),
    ],
    out_shape=[
        jax.ShapeDtypeStruct((N, H), _f32),
        jax.ShapeDtypeStruct((N, H), _f32),
    ],
)

def kernel(x, edge_index, W1l, b1l, W1r, W2l, b2l, W2r, Wm1, bm1, Wm2, bm2):
    src = edge_index[0].reshape(NW, NCH, C)
    dst = edge_index[1].reshape(NW, NCH, C)
    paug, r1 = _t1(x, W1l, W1r)
    tab1 = _seg80(paug, src, dst)
    p2, r2, inv8 = _t2(tab1, r1, b1l, W2l, W2r)
    tab2 = _seg64(p2, src, dst)
    a, b = _t3(tab2, r2, inv8, b2l, Wm1, bm1)
    return _edge(a, b, src, dst, Wm2.T, jnp.pad(bm2, (0, 14))).reshape(E, O)


# speedup vs baseline: 1.0691x; 1.0691x over previous
import jax
import jax.numpy as jnp
from jax import lax
from jax.experimental import pallas as pl
from jax.experimental.pallas import tpu as pltpu
from jax.experimental.pallas import tpu_sc as plsc

N = 10000
E = 320000
D = 128
H = 64
O = 2

NC = 2
NS = 16
NW = NC * NS
EPW = E // NW
C = 80
NCH = EPW // C
NP = 10240
RPT = NP // NS
ZB = 128

_f32 = jnp.float32


def _seg_kernel(width):
    mesh = plsc.VectorSubcoreMesh(core_axis_name="c", subcore_axis_name="s")
    gpr = width // 16

    def body(p_hbm, src_hbm, dst_hbm, out_hbm, idx_s, idx_d, rows, zbuf, table,
             isem, gsem, ssem):
        c = lax.axis_index("c")
        s = lax.axis_index("s")
        wid = c * NS + s

        pltpu.async_copy(src_hbm.at[wid], idx_s, isem)
        pltpu.async_copy(dst_hbm.at[wid], idx_d, isem)

        def zs(t, carry):
            zbuf[t // gpr, pl.ds((t % gpr) * 16, 16)] = jnp.zeros((16,), _f32)
            return carry

        lax.fori_loop(0, ZB * gpr, zs, 0)
        base_r = s * RPT
        for z in range(RPT // ZB):
            pltpu.sync_copy(zbuf, table.at[pl.ds(base_r + z * ZB, ZB)])
        pltpu.make_async_copy(src_hbm.at[wid], idx_s, isem).wait()
        pltpu.make_async_copy(dst_hbm.at[wid], idx_d, isem).wait()
        plsc.subcore_barrier()

        pltpu.async_copy(p_hbm.at[idx_s.at[0]], rows.at[0], gsem)

        def pair(m, carry):
            for b in range(2):
                j = 2 * m + b

                @pl.when(j < NCH)
                def _():
                    pltpu.make_async_copy(
                        p_hbm.at[pl.ds(0, C)], rows.at[b], gsem).wait()

                    @pl.when(j >= 1)
                    def _():
                        pltpu.make_async_copy(
                            rows.at[1 - b], table.at[pl.ds(0, C)], ssem).wait()

                    @pl.when(j + 1 < NCH)
                    def _():
                        pltpu.async_copy(
                            p_hbm.at[idx_s.at[j + 1]], rows.at[1 - b], gsem)

                    pltpu.async_copy(
                        rows.at[b], table.at[idx_d.at[j]], ssem, add=True)
            return carry

        lax.fori_loop(0, (NCH + 1) // 2, pair, 0)
        pltpu.make_async_copy(
            rows.at[(NCH - 1) % 2], table.at[pl.ds(0, C)], ssem).wait()
        plsc.subcore_barrier()
        pltpu.sync_copy(table.at[pl.ds(base_r, RPT)],
                        out_hbm.at[c, pl.ds(base_r, RPT)])

    return pl.kernel(
        body,
        out_type=jax.ShapeDtypeStruct((NC, NP, width), _f32),
        mesh=mesh,
        compiler_params=pltpu.CompilerParams(use_tc_tiling_on_sc=False),
        scratch_types=[
            pltpu.VMEM((NCH, C), jnp.int32),
            pltpu.VMEM((NCH, C), jnp.int32),
            pltpu.VMEM((2, C, width), _f32),
            pltpu.VMEM((ZB, width), _f32),
            pltpu.VMEM_SHARED((NP, width), _f32),
            pltpu.SemaphoreType.DMA,
            pltpu.SemaphoreType.DMA,
            pltpu.SemaphoreType.DMA,
        ],
    )


def _edge_kernel():
    mesh = plsc.VectorSubcoreMesh(core_axis_name="c", subcore_axis_name="s")

    def body(a_hbm, b_hbm, src_hbm, dst_hbm, wm2t_hbm, bm2_hbm, out_hbm,
             idx_s, idx_d, ra, rb, lg, wv, bv, gsem, wsem):
        c = lax.axis_index("c")
        s = lax.axis_index("s")
        wid = c * NS + s
        pltpu.sync_copy(wm2t_hbm, wv)
        pltpu.sync_copy(bm2_hbm, bv)
        pltpu.sync_copy(src_hbm.at[wid], idx_s)
        pltpu.sync_copy(dst_hbm.at[wid], idx_d)

        w0 = [wv[0, pl.ds(k * 16, 16)] for k in range(H // 16)]
        w1 = [wv[1, pl.ds(k * 16, 16)] for k in range(H // 16)]
        bvec = bv[pl.ds(0, 16)]
        b0 = bvec[0]
        b1 = bvec[1]
        io = lax.iota(jnp.int32, 16)
        io_lt2 = io < 2
        io_eq1 = io == 1

        pltpu.async_copy(a_hbm.at[idx_s.at[0]], ra.at[0], gsem)
        pltpu.async_copy(b_hbm.at[idx_d.at[0]], rb.at[0], gsem)

        def pair(m, carry):
            for b in range(2):
                j = 2 * m + b

                @pl.when(j < NCH)
                def _():
                    pltpu.make_async_copy(
                        a_hbm.at[pl.ds(0, C)], ra.at[b], gsem).wait()
                    pltpu.make_async_copy(
                        b_hbm.at[pl.ds(0, C)], rb.at[b], gsem).wait()

                    @pl.when(j >= 1)
                    def _():
                        pltpu.make_async_copy(
                            lg.at[1 - b], out_hbm.at[pl.ds(0, C)],
                            wsem).wait()

                    @pl.when(j + 1 < NCH)
                    def _():
                        pltpu.async_copy(
                            a_hbm.at[idx_s.at[j + 1]], ra.at[1 - b], gsem)
                        pltpu.async_copy(
                            b_hbm.at[idx_d.at[j + 1]], rb.at[1 - b], gsem)

                    rab = ra.at[b]
                    rbb = rb.at[b]
                    lgb = lg.at[b]

                    def vop(i, cc):
                        g = [jnp.maximum(rab[i, pl.ds(k * 16, 16)]
                                         + rbb[i, pl.ds(k * 16, 16)], 0.0)
                             for k in range(H // 16)]
                        s0 = g[0] * w0[0]
                        s1 = g[0] * w1[0]
                        for k in range(1, H // 16):
                            s0 = s0 + g[k] * w0[k]
                            s1 = s1 + g[k] * w1[k]
                        v = jnp.where(io_eq1, jnp.sum(s1) + b1,
                                      jnp.sum(s0) + b0)
                        plsc.store_scatter(lgb, [io * 0 + i, io], v,
                                           mask=io_lt2)
                        return cc

                    lax.fori_loop(0, C, vop, 0)
                    pltpu.async_copy(
                        lgb, out_hbm.at[pl.ds(wid * EPW + j * C, C)],
                        wsem)
            return carry

        lax.fori_loop(0, (NCH + 1) // 2, pair, 0)
        pltpu.make_async_copy(
            lg.at[(NCH - 1) % 2], out_hbm.at[pl.ds(0, C)], wsem).wait()

    return pl.kernel(
        body,
        out_type=jax.ShapeDtypeStruct((E, O), _f32),
        mesh=mesh,
        compiler_params=pltpu.CompilerParams(
            use_tc_tiling_on_sc=False, needs_layout_passes=False),
        scratch_types=[
            pltpu.VMEM((NCH, C), jnp.int32),
            pltpu.VMEM((NCH, C), jnp.int32),
            pltpu.VMEM((2, C, H), _f32),
            pltpu.VMEM((2, C, H), _f32),
            pltpu.VMEM((2, C, O), _f32),
            pltpu.VMEM((O, H), _f32),
            pltpu.VMEM((16,), _f32),
            pltpu.SemaphoreType.DMA,
            pltpu.SemaphoreType.DMA,
        ],
    )


_seg80 = _seg_kernel(H + 16)
_seg64 = _seg_kernel(H)
_edge = _edge_kernel()

RB = 2000


def _t1_body(x_ref, w1l_ref, w1r_ref, paug_ref, r1_ref):
    xb = x_ref[...]
    p1 = jnp.dot(xb, w1l_ref[...], preferred_element_type=_f32)
    paug_ref[...] = jnp.concatenate(
        [p1, jnp.ones((xb.shape[0], 16), _f32)], axis=1)
    r1_ref[...] = jnp.dot(xb, w1r_ref[...], preferred_element_type=_f32)


_t1 = pl.pallas_call(
    _t1_body,
    grid=(N // RB,),
    in_specs=[
        pl.BlockSpec((RB, D), lambda i: (i, 0)),
        pl.BlockSpec((D, H), lambda i: (0, 0)),
        pl.BlockSpec((D, H), lambda i: (0, 0)),
    ],
    out_specs=[
        pl.BlockSpec((RB, H + 16), lambda i: (i, 0)),
        pl.BlockSpec((RB, H), lambda i: (i, 0)),
    ],
    out_shape=[
        jax.ShapeDtypeStruct((N, H + 16), _f32),
        jax.ShapeDtypeStruct((N, H), _f32),
    ],
)


def _t2_body(tab_ref, r1_ref, b1l_ref, w2l_ref, w2r_ref, p2_ref, r2_ref, inv_ref):
    tab = tab_ref[...]
    agg = tab[0, :, :H] + tab[1, :, :H]
    cnt = tab[0, :, H:H + 1] + tab[1, :, H:H + 1]
    inv = 1.0 / jnp.maximum(cnt, 1.0)
    h1 = jnp.maximum(agg * inv + b1l_ref[...][None, :] + r1_ref[...], 0.0)
    p2_ref[...] = jnp.dot(h1, w2l_ref[...], preferred_element_type=_f32)
    r2_ref[...] = jnp.dot(h1, w2r_ref[...], preferred_element_type=_f32)
    inv_ref[...] = jnp.broadcast_to(inv, (inv.shape[0], 8))


_t2 = pl.pallas_call(
    _t2_body,
    grid=(N // RB,),
    in_specs=[
        pl.BlockSpec((NC, RB, H + 16), lambda i: (0, i, 0)),
        pl.BlockSpec((RB, H), lambda i: (i, 0)),
        pl.BlockSpec((H,), lambda i: (0,)),
        pl.BlockSpec((H, H), lambda i: (0, 0)),
        pl.BlockSpec((H, H), lambda i: (0, 0)),
    ],
    out_specs=[
        pl.BlockSpec((RB, H), lambda i: (i, 0)),
        pl.BlockSpec((RB, H), lambda i: (i, 0)),
        pl.BlockSpec((RB, 8), lambda i: (i, 0)),
    ],
    out_shape=[
        jax.ShapeDtypeStruct((N, H), _f32),
        jax.ShapeDtypeStruct((N, H), _f32),
        jax.ShapeDtypeStruct((N, 8), _f32),
    ],
)


def _t3_body(tab_ref, r2_ref, inv_ref, b2l_ref, wm1_ref, bm1_ref, a_ref, b_ref):
    tab = tab_ref[...]
    agg = tab[0] + tab[1]
    inv = inv_ref[...][:, :1]
    h2 = jnp.maximum(agg * inv + b2l_ref[...][None, :] + r2_ref[...], 0.0)
    wm1 = wm1_ref[...]
    a_ref[...] = jnp.dot(h2, wm1[:H], preferred_element_type=_f32) \
        + bm1_ref[...][None, :]
    b_ref[...] = jnp.dot(h2, wm1[H:], preferred_element_type=_f32)


_t3 = pl.pallas_call(
    _t3_body,
    grid=(N // RB,),
    in_specs=[
        pl.BlockSpec((NC, RB, H), lambda i: (0, i, 0)),
        pl.BlockSpec((RB, H), lambda i: (i, 0)),
        pl.BlockSpec((RB, 8), lambda i: (i, 0)),
        pl.BlockSpec((H,), lambda i: (0,)),
        pl.BlockSpec((2 * H, H), lambda i: (0, 0)),
        pl.BlockSpec((H,), lambda i: (0,)),
    ],
    out_specs=[
        pl.BlockSpec((RB, H), lambda i: (i, 0)),
        pl.BlockSpec((RB, H), lambda i: (i, 0)),
    ],
    out_shape=[
        jax.ShapeDtypeStruct((N, H), _f32),
        jax.ShapeDtypeStruct((N, H), _f32),
    ],
)

def kernel(x, edge_index, W1l, b1l, W1r, W2l, b2l, W2r, Wm1, bm1, Wm2, bm2):
    src = edge_index[0].reshape(NW, NCH, C)
    dst = edge_index[1].reshape(NW, NCH, C)
    paug, r1 = _t1(x, W1l, W1r)
    tab1 = _seg80(paug, src, dst)
    p2, r2, inv8 = _t2(tab1, r1, b1l, W2l, W2r)
    tab2 = _seg64(p2, src, dst)
    a, b = _t3(tab2, r2, inv8, b2l, Wm1, bm1)
    return _edge(a, b, src, dst, Wm2.T, jnp.pad(bm2, (0, 14)))


# speedup vs baseline: 1.1523x; 1.0779x over previous
import jax
import jax.numpy as jnp
from jax import lax
from jax.experimental import pallas as pl
from jax.experimental.pallas import tpu as pltpu
from jax.experimental.pallas import tpu_sc as plsc

N = 10000
E = 320000
D = 128
H = 64
O = 2

NC = 2
NS = 16
NW = NC * NS
EPW = E // NW
C = 80
NCH = EPW // C
NP = 10240
RPT = NP // NS
ZB = 128

_f32 = jnp.float32


def _seg_kernel(width):
    mesh = plsc.VectorSubcoreMesh(core_axis_name="c", subcore_axis_name="s")
    gpr = width // 16

    def body(p_hbm, src_hbm, dst_hbm, out_hbm, idx_s, idx_d, rows, zbuf, table,
             isem, gsem, ssem):
        c = lax.axis_index("c")
        s = lax.axis_index("s")
        wid = c * NS + s

        pltpu.async_copy(src_hbm.at[wid], idx_s, isem)
        pltpu.async_copy(dst_hbm.at[wid], idx_d, isem)

        def zs(t, carry):
            zbuf[t // gpr, pl.ds((t % gpr) * 16, 16)] = jnp.zeros((16,), _f32)
            return carry

        lax.fori_loop(0, ZB * gpr, zs, 0)
        base_r = s * RPT
        for z in range(RPT // ZB):
            pltpu.sync_copy(zbuf, table.at[pl.ds(base_r + z * ZB, ZB)])
        pltpu.make_async_copy(src_hbm.at[wid], idx_s, isem).wait()
        pltpu.make_async_copy(dst_hbm.at[wid], idx_d, isem).wait()
        plsc.subcore_barrier()

        pltpu.async_copy(p_hbm.at[idx_s.at[0]], rows.at[0], gsem)

        def pair(m, carry):
            for b in range(2):
                j = 2 * m + b

                @pl.when(j < NCH)
                def _():
                    pltpu.make_async_copy(
                        p_hbm.at[pl.ds(0, C)], rows.at[b], gsem).wait()

                    @pl.when(j >= 1)
                    def _():
                        pltpu.make_async_copy(
                            rows.at[1 - b], table.at[pl.ds(0, C)], ssem).wait()

                    @pl.when(j + 1 < NCH)
                    def _():
                        pltpu.async_copy(
                            p_hbm.at[idx_s.at[j + 1]], rows.at[1 - b], gsem)

                    pltpu.async_copy(
                        rows.at[b], table.at[idx_d.at[j]], ssem, add=True)
            return carry

        lax.fori_loop(0, (NCH + 1) // 2, pair, 0)
        pltpu.make_async_copy(
            rows.at[(NCH - 1) % 2], table.at[pl.ds(0, C)], ssem).wait()
        plsc.subcore_barrier()
        pltpu.sync_copy(table.at[pl.ds(base_r, RPT)],
                        out_hbm.at[c, pl.ds(base_r, RPT)])

    return pl.kernel(
        body,
        out_type=jax.ShapeDtypeStruct((NC, NP, width), _f32),
        mesh=mesh,
        compiler_params=pltpu.CompilerParams(use_tc_tiling_on_sc=False),
        scratch_types=[
            pltpu.VMEM((NCH, C), jnp.int32),
            pltpu.VMEM((NCH, C), jnp.int32),
            pltpu.VMEM((2, C, width), _f32),
            pltpu.VMEM((ZB, width), _f32),
            pltpu.VMEM_SHARED((NP, width), _f32),
            pltpu.SemaphoreType.DMA,
            pltpu.SemaphoreType.DMA,
            pltpu.SemaphoreType.DMA,
        ],
    )


def _edge_kernel():
    mesh = plsc.VectorSubcoreMesh(core_axis_name="c", subcore_axis_name="s")

    def body(a_hbm, b_hbm, src_hbm, dst_hbm, wm2t_hbm, bm2_hbm, out_hbm,
             idx_s, idx_d, ra, rb, lg, wv, bv, gsem, wsem):
        c = lax.axis_index("c")
        s = lax.axis_index("s")
        wid = c * NS + s
        pltpu.sync_copy(wm2t_hbm, wv)
        pltpu.sync_copy(bm2_hbm, bv)
        pltpu.sync_copy(src_hbm.at[wid], idx_s)
        pltpu.sync_copy(dst_hbm.at[wid], idx_d)

        w0 = [wv[0, pl.ds(k * 16, 16)] for k in range(H // 16)]
        w1 = [wv[1, pl.ds(k * 16, 16)] for k in range(H // 16)]
        bvec = bv[pl.ds(0, 16)]
        b0 = bvec[0]
        b1 = bvec[1]
        io = lax.iota(jnp.int32, 16)
        io_lt2 = io < 2
        io_lt8 = io < 8
        px8 = io ^ 8
        px4 = io ^ 4
        px2 = io ^ 2
        px1 = io ^ 1
        psel = jnp.where(io_eq1_first := (io == 1), 8, 0)

        pltpu.async_copy(a_hbm.at[idx_s.at[0]], ra.at[0], gsem)
        pltpu.async_copy(b_hbm.at[idx_d.at[0]], rb.at[0], gsem)

        def pair(m, carry):
            for b in range(2):
                j = 2 * m + b

                @pl.when(j < NCH)
                def _():
                    pltpu.make_async_copy(
                        a_hbm.at[pl.ds(0, C)], ra.at[b], gsem).wait()
                    pltpu.make_async_copy(
                        b_hbm.at[pl.ds(0, C)], rb.at[b], gsem).wait()

                    @pl.when(j >= 1)
                    def _():
                        pltpu.make_async_copy(
                            lg.at[1 - b], out_hbm.at[pl.ds(0, C)],
                            wsem).wait()

                    @pl.when(j + 1 < NCH)
                    def _():
                        pltpu.async_copy(
                            a_hbm.at[idx_s.at[j + 1]], ra.at[1 - b], gsem)
                        pltpu.async_copy(
                            b_hbm.at[idx_d.at[j + 1]], rb.at[1 - b], gsem)

                    rab = ra.at[b]
                    rbb = rb.at[b]
                    lgb = lg.at[b]

                    def tk(v, p):
                        return lax.gather(
                            v, p[:, None],
                            dimension_numbers=lax.GatherDimensionNumbers(
                                offset_dims=(), collapsed_slice_dims=(0,),
                                start_index_map=(0,)),
                            slice_sizes=(1,),
                            mode=lax.GatherScatterMode.PROMISE_IN_BOUNDS)

                    def vop(i, cc):
                        g = [jnp.maximum(rab[i, pl.ds(k * 16, 16)]
                                         + rbb[i, pl.ds(k * 16, 16)], 0.0)
                             for k in range(H // 16)]
                        s0 = g[0] * w0[0]
                        s1 = g[0] * w1[0]
                        for k in range(1, H // 16):
                            s0 = s0 + g[k] * w0[k]
                            s1 = s1 + g[k] * w1[k]
                        f0 = s0 + tk(s0, px8)
                        f1 = s1 + tk(s1, px8)
                        m = jnp.where(io_lt8, f0, tk(f1, px8))
                        m = m + tk(m, px4)
                        m = m + tk(m, px2)
                        m = m + tk(m, px1)
                        v = tk(m, psel) + bvec
                        plsc.store_scatter(lgb, [io * 0 + i, io], v,
                                           mask=io_lt2)
                        return cc

                    lax.fori_loop(0, C, vop, 0)
                    pltpu.async_copy(
                        lgb, out_hbm.at[pl.ds(wid * EPW + j * C, C)],
                        wsem)
            return carry

        lax.fori_loop(0, (NCH + 1) // 2, pair, 0)
        pltpu.make_async_copy(
            lg.at[(NCH - 1) % 2], out_hbm.at[pl.ds(0, C)], wsem).wait()

    return pl.kernel(
        body,
        out_type=jax.ShapeDtypeStruct((E, O), _f32),
        mesh=mesh,
        compiler_params=pltpu.CompilerParams(
            use_tc_tiling_on_sc=False, needs_layout_passes=False),
        scratch_types=[
            pltpu.VMEM((NCH, C), jnp.int32),
            pltpu.VMEM((NCH, C), jnp.int32),
            pltpu.VMEM((2, C, H), _f32),
            pltpu.VMEM((2, C, H), _f32),
            pltpu.VMEM((2, C, O), _f32),
            pltpu.VMEM((O, H), _f32),
            pltpu.VMEM((16,), _f32),
            pltpu.SemaphoreType.DMA,
            pltpu.SemaphoreType.DMA,
        ],
    )


_seg80 = _seg_kernel(H + 16)
_seg64 = _seg_kernel(H)
_edge = _edge_kernel()

RB = 2000


def _t1_body(x_ref, w1l_ref, w1r_ref, paug_ref, r1_ref):
    xb = x_ref[...]
    p1 = jnp.dot(xb, w1l_ref[...], preferred_element_type=_f32)
    paug_ref[...] = jnp.concatenate(
        [p1, jnp.ones((xb.shape[0], 16), _f32)], axis=1)
    r1_ref[...] = jnp.dot(xb, w1r_ref[...], preferred_element_type=_f32)


_t1 = pl.pallas_call(
    _t1_body,
    grid=(N // RB,),
    in_specs=[
        pl.BlockSpec((RB, D), lambda i: (i, 0)),
        pl.BlockSpec((D, H), lambda i: (0, 0)),
        pl.BlockSpec((D, H), lambda i: (0, 0)),
    ],
    out_specs=[
        pl.BlockSpec((RB, H + 16), lambda i: (i, 0)),
        pl.BlockSpec((RB, H), lambda i: (i, 0)),
    ],
    out_shape=[
        jax.ShapeDtypeStruct((N, H + 16), _f32),
        jax.ShapeDtypeStruct((N, H), _f32),
    ],
)


def _t2_body(tab_ref, r1_ref, b1l_ref, w2l_ref, w2r_ref, p2_ref, r2_ref, inv_ref):
    tab = tab_ref[...]
    agg = tab[0, :, :H] + tab[1, :, :H]
    cnt = tab[0, :, H:H + 1] + tab[1, :, H:H + 1]
    inv = 1.0 / jnp.maximum(cnt, 1.0)
    h1 = jnp.maximum(agg * inv + b1l_ref[...][None, :] + r1_ref[...], 0.0)
    p2_ref[...] = jnp.dot(h1, w2l_ref[...], preferred_element_type=_f32)
    r2_ref[...] = jnp.dot(h1, w2r_ref[...], preferred_element_type=_f32)
    inv_ref[...] = jnp.broadcast_to(inv, (inv.shape[0], 8))


_t2 = pl.pallas_call(
    _t2_body,
    grid=(N // RB,),
    in_specs=[
        pl.BlockSpec((NC, RB, H + 16), lambda i: (0, i, 0)),
        pl.BlockSpec((RB, H), lambda i: (i, 0)),
        pl.BlockSpec((H,), lambda i: (0,)),
        pl.BlockSpec((H, H), lambda i: (0, 0)),
        pl.BlockSpec((H, H), lambda i: (0, 0)),
    ],
    out_specs=[
        pl.BlockSpec((RB, H), lambda i: (i, 0)),
        pl.BlockSpec((RB, H), lambda i: (i, 0)),
        pl.BlockSpec((RB, 8), lambda i: (i, 0)),
    ],
    out_shape=[
        jax.ShapeDtypeStruct((N, H), _f32),
        jax.ShapeDtypeStruct((N, H), _f32),
        jax.ShapeDtypeStruct((N, 8), _f32),
    ],
)


def _t3_body(tab_ref, r2_ref, inv_ref, b2l_ref, wm1_ref, bm1_ref, a_ref, b_ref):
    tab = tab_ref[...]
    agg = tab[0] + tab[1]
    inv = inv_ref[...][:, :1]
    h2 = jnp.maximum(agg * inv + b2l_ref[...][None, :] + r2_ref[...], 0.0)
    wm1 = wm1_ref[...]
    a_ref[...] = jnp.dot(h2, wm1[:H], preferred_element_type=_f32) \
        + bm1_ref[...][None, :]
    b_ref[...] = jnp.dot(h2, wm1[H:], preferred_element_type=_f32)


_t3 = pl.pallas_call(
    _t3_body,
    grid=(N // RB,),
    in_specs=[
        pl.BlockSpec((NC, RB, H), lambda i: (0, i, 0)),
        pl.BlockSpec((RB, H), lambda i: (i, 0)),
        pl.BlockSpec((RB, 8), lambda i: (i, 0)),
        pl.BlockSpec((H,), lambda i: (0,)),
        pl.BlockSpec((2 * H, H), lambda i: (0, 0)),
        pl.BlockSpec((H,), lambda i: (0,)),
    ],
    out_specs=[
        pl.BlockSpec((RB, H), lambda i: (i, 0)),
        pl.BlockSpec((RB, H), lambda i: (i, 0)),
    ],
    out_shape=[
        jax.ShapeDtypeStruct((N, H), _f32),
        jax.ShapeDtypeStruct((N, H), _f32),
    ],
)

def kernel(x, edge_index, W1l, b1l, W1r, W2l, b2l, W2r, Wm1, bm1, Wm2, bm2):
    src = edge_index[0].reshape(NW, NCH, C)
    dst = edge_index[1].reshape(NW, NCH, C)
    paug, r1 = _t1(x, W1l, W1r)
    tab1 = _seg80(paug, src, dst)
    p2, r2, inv8 = _t2(tab1, r1, b1l, W2l, W2r)
    tab2 = _seg64(p2, src, dst)
    a, b = _t3(tab2, r2, inv8, b2l, Wm1, bm1)
    return _edge(a, b, src, dst, Wm2.T, jnp.pad(bm2, (0, 14)))


# speedup vs baseline: 1.2553x; 1.0894x over previous
import jax
import jax.numpy as jnp
from jax import lax
from jax.experimental import pallas as pl
from jax.experimental.pallas import tpu as pltpu
from jax.experimental.pallas import tpu_sc as plsc

N = 10000
E = 320000
D = 128
H = 64
O = 2

NC = 2
NS = 16
NW = NC * NS
EPW = E // NW
C = 80
NCH = EPW // C
NP = 10240
RPT = NP // NS
ZB = 128

_f32 = jnp.float32


def _seg_kernel(width):
    mesh = plsc.VectorSubcoreMesh(core_axis_name="c", subcore_axis_name="s")
    gpr = width // 16

    def body(p_hbm, src_hbm, dst_hbm, out_hbm, idx_s, idx_d, rows, zbuf, table,
             isem, gsem, ssem):
        c = lax.axis_index("c")
        s = lax.axis_index("s")
        wid = c * NS + s

        pltpu.async_copy(src_hbm.at[wid], idx_s, isem)
        pltpu.async_copy(dst_hbm.at[wid], idx_d, isem)

        def zs(t, carry):
            zbuf[t // gpr, pl.ds((t % gpr) * 16, 16)] = jnp.zeros((16,), _f32)
            return carry

        lax.fori_loop(0, ZB * gpr, zs, 0)
        base_r = s * RPT
        for z in range(RPT // ZB):
            pltpu.sync_copy(zbuf, table.at[pl.ds(base_r + z * ZB, ZB)])
        pltpu.make_async_copy(src_hbm.at[wid], idx_s, isem).wait()
        pltpu.make_async_copy(dst_hbm.at[wid], idx_d, isem).wait()
        plsc.subcore_barrier()

        pltpu.async_copy(p_hbm.at[idx_s.at[0]], rows.at[0], gsem)

        def pair(m, carry):
            for b in range(2):
                j = 2 * m + b

                @pl.when(j < NCH)
                def _():
                    pltpu.make_async_copy(
                        p_hbm.at[pl.ds(0, C)], rows.at[b], gsem).wait()

                    @pl.when(j >= 1)
                    def _():
                        pltpu.make_async_copy(
                            rows.at[1 - b], table.at[pl.ds(0, C)], ssem).wait()

                    @pl.when(j + 1 < NCH)
                    def _():
                        pltpu.async_copy(
                            p_hbm.at[idx_s.at[j + 1]], rows.at[1 - b], gsem)

                    pltpu.async_copy(
                        rows.at[b], table.at[idx_d.at[j]], ssem, add=True)
            return carry

        lax.fori_loop(0, (NCH + 1) // 2, pair, 0)
        pltpu.make_async_copy(
            rows.at[(NCH - 1) % 2], table.at[pl.ds(0, C)], ssem).wait()
        plsc.subcore_barrier()
        pltpu.sync_copy(table.at[pl.ds(base_r, RPT)],
                        out_hbm.at[c, pl.ds(base_r, RPT)])

    return pl.kernel(
        body,
        out_type=jax.ShapeDtypeStruct((NC, NP, width), _f32),
        mesh=mesh,
        compiler_params=pltpu.CompilerParams(use_tc_tiling_on_sc=False),
        scratch_types=[
            pltpu.VMEM((NCH, C), jnp.int32),
            pltpu.VMEM((NCH, C), jnp.int32),
            pltpu.VMEM((2, C, width), _f32),
            pltpu.VMEM((ZB, width), _f32),
            pltpu.VMEM_SHARED((NP, width), _f32),
            pltpu.SemaphoreType.DMA,
            pltpu.SemaphoreType.DMA,
            pltpu.SemaphoreType.DMA,
        ],
    )


def _edge_kernel():
    mesh = plsc.VectorSubcoreMesh(core_axis_name="c", subcore_axis_name="s")

    def body(ab_hbm, src_hbm, dst_hbm, wm2t_hbm, bm2_hbm, out_hbm,
             idx_s, idx_d, ra, rb, lg, wv, bv, gsem, wsem):
        c = lax.axis_index("c")
        s = lax.axis_index("s")
        wid = c * NS + s
        pltpu.sync_copy(wm2t_hbm, wv)
        pltpu.sync_copy(bm2_hbm, bv)
        pltpu.sync_copy(src_hbm.at[wid], idx_s)
        pltpu.sync_copy(dst_hbm.at[wid], idx_d)

        w0 = [wv[0, pl.ds(k * 16, 16)] for k in range(H // 16)]
        w1 = [wv[1, pl.ds(k * 16, 16)] for k in range(H // 16)]
        bvec = bv[pl.ds(0, 16)]
        b0 = bvec[0]
        b1 = bvec[1]
        io = lax.iota(jnp.int32, 16)
        io_lt2 = io < 2
        io_lt8 = io < 8
        px8 = io ^ 8
        px4 = io ^ 4
        px2 = io ^ 2
        px1 = io ^ 1
        psel = jnp.where(io_eq1_first := (io == 1), 8, 0)

        pltpu.async_copy(ab_hbm.at[idx_s.at[0]], ra.at[0], gsem)
        pltpu.async_copy(ab_hbm.at[idx_d.at[0]], rb.at[0], gsem)

        def pair(m, carry):
            for b in range(2):
                j = 2 * m + b

                @pl.when(j < NCH)
                def _():
                    pltpu.make_async_copy(
                        ab_hbm.at[pl.ds(0, C)], ra.at[b], gsem).wait()
                    pltpu.make_async_copy(
                        ab_hbm.at[pl.ds(0, C)], rb.at[b], gsem).wait()

                    @pl.when(j >= 1)
                    def _():
                        pltpu.make_async_copy(
                            lg.at[1 - b], out_hbm.at[pl.ds(0, C)],
                            wsem).wait()

                    @pl.when(j + 1 < NCH)
                    def _():
                        pltpu.async_copy(
                            ab_hbm.at[idx_s.at[j + 1]], ra.at[1 - b], gsem)
                        pltpu.async_copy(
                            ab_hbm.at[idx_d.at[j + 1]], rb.at[1 - b], gsem)

                    rab = ra.at[b]
                    rbb = rb.at[b]
                    lgb = lg.at[b]

                    def tk(v, p):
                        return lax.gather(
                            v, p[:, None],
                            dimension_numbers=lax.GatherDimensionNumbers(
                                offset_dims=(), collapsed_slice_dims=(0,),
                                start_index_map=(0,)),
                            slice_sizes=(1,),
                            mode=lax.GatherScatterMode.PROMISE_IN_BOUNDS)

                    def vop(i, cc):
                        g = [jnp.maximum(rab[i, pl.ds(k * 16, 16)]
                                         + rbb[i, pl.ds(H + k * 16, 16)], 0.0)
                             for k in range(H // 16)]
                        s0 = g[0] * w0[0]
                        s1 = g[0] * w1[0]
                        for k in range(1, H // 16):
                            s0 = s0 + g[k] * w0[k]
                            s1 = s1 + g[k] * w1[k]
                        f0 = s0 + tk(s0, px8)
                        f1 = s1 + tk(s1, px8)
                        m = jnp.where(io_lt8, f0, tk(f1, px8))
                        m = m + tk(m, px4)
                        m = m + tk(m, px2)
                        m = m + tk(m, px1)
                        v = tk(m, psel) + bvec
                        plsc.store_scatter(lgb, [io * 0 + i, io], v,
                                           mask=io_lt2)
                        return cc

                    lax.fori_loop(0, C, vop, 0)
                    pltpu.async_copy(
                        lgb, out_hbm.at[pl.ds(wid * EPW + j * C, C)],
                        wsem)
            return carry

        lax.fori_loop(0, (NCH + 1) // 2, pair, 0)
        pltpu.make_async_copy(
            lg.at[(NCH - 1) % 2], out_hbm.at[pl.ds(0, C)], wsem).wait()

    return pl.kernel(
        body,
        out_type=jax.ShapeDtypeStruct((E, O), _f32),
        mesh=mesh,
        compiler_params=pltpu.CompilerParams(needs_layout_passes=False),
        scratch_types=[
            pltpu.VMEM((NCH, C), jnp.int32),
            pltpu.VMEM((NCH, C), jnp.int32),
            pltpu.VMEM((2, C, 2 * H), _f32),
            pltpu.VMEM((2, C, 2 * H), _f32),
            pltpu.VMEM((2, C, O), _f32),
            pltpu.VMEM((O, H), _f32),
            pltpu.VMEM((16,), _f32),
            pltpu.SemaphoreType.DMA,
            pltpu.SemaphoreType.DMA,
        ],
    )


_seg80 = _seg_kernel(H + 16)
_seg64 = _seg_kernel(H)
_edge = _edge_kernel()

RB = 2000


def _t1_body(x_ref, w1l_ref, w1r_ref, paug_ref, r1_ref):
    xb = x_ref[...]
    p1 = jnp.dot(xb, w1l_ref[...], preferred_element_type=_f32)
    paug_ref[...] = jnp.concatenate(
        [p1, jnp.ones((xb.shape[0], 16), _f32)], axis=1)
    r1_ref[...] = jnp.dot(xb, w1r_ref[...], preferred_element_type=_f32)


_t1 = pl.pallas_call(
    _t1_body,
    grid=(N // RB,),
    in_specs=[
        pl.BlockSpec((RB, D), lambda i: (i, 0)),
        pl.BlockSpec((D, H), lambda i: (0, 0)),
        pl.BlockSpec((D, H), lambda i: (0, 0)),
    ],
    out_specs=[
        pl.BlockSpec((RB, H + 16), lambda i: (i, 0)),
        pl.BlockSpec((RB, H), lambda i: (i, 0)),
    ],
    out_shape=[
        jax.ShapeDtypeStruct((N, H + 16), _f32),
        jax.ShapeDtypeStruct((N, H), _f32),
    ],
)


def _t2_body(tab_ref, r1_ref, b1l_ref, w2l_ref, w2r_ref, p2_ref, r2_ref, inv_ref):
    tab = tab_ref[...]
    agg = tab[0, :, :H] + tab[1, :, :H]
    cnt = tab[0, :, H:H + 1] + tab[1, :, H:H + 1]
    inv = 1.0 / jnp.maximum(cnt, 1.0)
    h1 = jnp.maximum(agg * inv + b1l_ref[...][None, :] + r1_ref[...], 0.0)
    p2_ref[...] = jnp.dot(h1, w2l_ref[...], preferred_element_type=_f32)
    r2_ref[...] = jnp.dot(h1, w2r_ref[...], preferred_element_type=_f32)
    inv_ref[...] = jnp.broadcast_to(inv, (inv.shape[0], 8))


_t2 = pl.pallas_call(
    _t2_body,
    grid=(N // RB,),
    in_specs=[
        pl.BlockSpec((NC, RB, H + 16), lambda i: (0, i, 0)),
        pl.BlockSpec((RB, H), lambda i: (i, 0)),
        pl.BlockSpec((H,), lambda i: (0,)),
        pl.BlockSpec((H, H), lambda i: (0, 0)),
        pl.BlockSpec((H, H), lambda i: (0, 0)),
    ],
    out_specs=[
        pl.BlockSpec((RB, H), lambda i: (i, 0)),
        pl.BlockSpec((RB, H), lambda i: (i, 0)),
        pl.BlockSpec((RB, 8), lambda i: (i, 0)),
    ],
    out_shape=[
        jax.ShapeDtypeStruct((N, H), _f32),
        jax.ShapeDtypeStruct((N, H), _f32),
        jax.ShapeDtypeStruct((N, 8), _f32),
    ],
)


def _t3_body(tab_ref, r2_ref, inv_ref, b2l_ref, wm1_ref, bm1_ref, ab_ref):
    tab = tab_ref[...]
    agg = tab[0] + tab[1]
    inv = inv_ref[...][:, :1]
    h2 = jnp.maximum(agg * inv + b2l_ref[...][None, :] + r2_ref[...], 0.0)
    wm1 = wm1_ref[...]
    a = jnp.dot(h2, wm1[:H], preferred_element_type=_f32) \
        + bm1_ref[...][None, :]
    b = jnp.dot(h2, wm1[H:], preferred_element_type=_f32)
    ab_ref[...] = jnp.concatenate([a, b], axis=1)


_t3 = pl.pallas_call(
    _t3_body,
    grid=(N // RB,),
    in_specs=[
        pl.BlockSpec((NC, RB, H), lambda i: (0, i, 0)),
        pl.BlockSpec((RB, H), lambda i: (i, 0)),
        pl.BlockSpec((RB, 8), lambda i: (i, 0)),
        pl.BlockSpec((H,), lambda i: (0,)),
        pl.BlockSpec((2 * H, H), lambda i: (0, 0)),
        pl.BlockSpec((H,), lambda i: (0,)),
    ],
    out_specs=pl.BlockSpec((RB, 2 * H), lambda i: (i, 0)),
    out_shape=jax.ShapeDtypeStruct((N, 2 * H), _f32),
)


def kernel(x, edge_index, W1l, b1l, W1r, W2l, b2l, W2r, Wm1, bm1, Wm2, bm2):
    src = edge_index[0].reshape(NW, NCH, C)
    dst = edge_index[1].reshape(NW, NCH, C)
    paug, r1 = _t1(x, W1l, W1r)
    tab1 = _seg80(paug, src, dst)
    p2, r2, inv8 = _t2(tab1, r1, b1l, W2l, W2r)
    tab2 = _seg64(p2, src, dst)
    ab = _t3(tab2, r2, inv8, b2l, Wm1, bm1)
    return _edge(ab, src, dst, Wm2.T, jnp.pad(bm2, (0, 14)))


# speedup vs baseline: 1.5267x; 1.2162x over previous
import jax
import jax.numpy as jnp
from jax import lax
from jax.experimental import pallas as pl
from jax.experimental.pallas import tpu as pltpu
from jax.experimental.pallas import tpu_sc as plsc

N = 10000
E = 320000
D = 128
H = 64
O = 2

NC = 2
NS = 16
NW = NC * NS
EPW = E // NW
C = 80
NCH = EPW // C
NP = 10240
RPT = NP // NS
ZB = 128

_f32 = jnp.float32


def _seg_kernel(width):
    mesh = plsc.VectorSubcoreMesh(core_axis_name="c", subcore_axis_name="s")
    gpr = width // 16

    def body(p_hbm, src_hbm, dst_hbm, out_hbm, idx_s, idx_d, rows, zbuf, table,
             isem, gsem, ssem):
        c = lax.axis_index("c")
        s = lax.axis_index("s")
        wid = c * NS + s

        pltpu.async_copy(src_hbm.at[wid], idx_s, isem)
        pltpu.async_copy(dst_hbm.at[wid], idx_d, isem)

        def zs(t, carry):
            zbuf[t // gpr, pl.ds((t % gpr) * 16, 16)] = jnp.zeros((16,), _f32)
            return carry

        lax.fori_loop(0, ZB * gpr, zs, 0)
        base_r = s * RPT
        for z in range(RPT // ZB):
            pltpu.sync_copy(zbuf, table.at[pl.ds(base_r + z * ZB, ZB)])
        pltpu.make_async_copy(src_hbm.at[wid], idx_s, isem).wait()
        pltpu.make_async_copy(dst_hbm.at[wid], idx_d, isem).wait()
        plsc.subcore_barrier()

        pltpu.async_copy(p_hbm.at[idx_s.at[0]], rows.at[0], gsem)
        pltpu.async_copy(p_hbm.at[idx_s.at[1]], rows.at[1], gsem)

        def trip(m, carry):
            for b in range(3):
                j = 3 * m + b

                @pl.when(j < NCH)
                def _():
                    pltpu.make_async_copy(
                        p_hbm.at[pl.ds(0, C)], rows.at[b], gsem).wait()

                    @pl.when(j >= 1)
                    def _():
                        pltpu.make_async_copy(
                            rows.at[b], table.at[pl.ds(0, C)], ssem).wait()

                    @pl.when(j + 2 < NCH)
                    def _():
                        pltpu.async_copy(
                            p_hbm.at[idx_s.at[j + 2]],
                            rows.at[(b + 2) % 3], gsem)

                    pltpu.async_copy(
                        rows.at[b], table.at[idx_d.at[j]], ssem, add=True)
            return carry

        lax.fori_loop(0, (NCH + 2) // 3, trip, 0)
        pltpu.make_async_copy(
            rows.at[0], table.at[pl.ds(0, C)], ssem).wait()
        plsc.subcore_barrier()
        pltpu.sync_copy(table.at[pl.ds(base_r, RPT)],
                        out_hbm.at[c, pl.ds(base_r, RPT)])

    return pl.kernel(
        body,
        out_type=jax.ShapeDtypeStruct((NC, NP, width), _f32),
        mesh=mesh,
        compiler_params=pltpu.CompilerParams(use_tc_tiling_on_sc=False),
        scratch_types=[
            pltpu.VMEM((NCH, C), jnp.int32),
            pltpu.VMEM((NCH, C), jnp.int32),
            pltpu.VMEM((3, C, width), _f32),
            pltpu.VMEM((ZB, width), _f32),
            pltpu.VMEM_SHARED((NP, width), _f32),
            pltpu.SemaphoreType.DMA,
            pltpu.SemaphoreType.DMA,
            pltpu.SemaphoreType.DMA,
        ],
    )


def _edge_kernel():
    mesh = plsc.VectorSubcoreMesh(core_axis_name="c", subcore_axis_name="s")

    def body(ab_hbm, src_hbm, dst_hbm, wm2t_hbm, bm2_hbm, out_hbm,
             idx_s, idx_d, ra, rb, lg, wv, bv, gsem, wsem):
        c = lax.axis_index("c")
        s = lax.axis_index("s")
        wid = c * NS + s
        pltpu.sync_copy(wm2t_hbm, wv)
        pltpu.sync_copy(bm2_hbm, bv)
        pltpu.sync_copy(src_hbm.at[wid], idx_s)
        pltpu.sync_copy(dst_hbm.at[wid], idx_d)

        w0 = [wv[0, pl.ds(k * 16, 16)] for k in range(H // 16)]
        w1 = [wv[1, pl.ds(k * 16, 16)] for k in range(H // 16)]
        bvec = bv[pl.ds(0, 16)]
        b0 = bvec[0]
        b1 = bvec[1]
        io = lax.iota(jnp.int32, 16)
        io_lt2 = io < 2
        io_lt8 = io < 8
        px8 = io ^ 8
        px4 = io ^ 4
        px2 = io ^ 2
        px1 = io ^ 1
        psel = jnp.where(io_eq1_first := (io == 1), 8, 0)

        pltpu.async_copy(ab_hbm.at[idx_s.at[0]], ra.at[0], gsem)
        pltpu.async_copy(ab_hbm.at[idx_d.at[0]], rb.at[0], gsem)
        pltpu.async_copy(ab_hbm.at[idx_s.at[1]], ra.at[1], gsem)
        pltpu.async_copy(ab_hbm.at[idx_d.at[1]], rb.at[1], gsem)

        def pair(m, carry):
            for b in range(3):
                j = 3 * m + b

                @pl.when(j < NCH)
                def _():
                    pltpu.make_async_copy(
                        ab_hbm.at[pl.ds(0, C)], ra.at[b], gsem).wait()
                    pltpu.make_async_copy(
                        ab_hbm.at[pl.ds(0, C)], rb.at[b], gsem).wait()

                    @pl.when(j >= 3)
                    def _():
                        pltpu.make_async_copy(
                            lg.at[b], out_hbm.at[pl.ds(0, C)],
                            wsem).wait()

                    @pl.when(j + 2 < NCH)
                    def _():
                        pltpu.async_copy(
                            ab_hbm.at[idx_s.at[j + 2]], ra.at[(b + 2) % 3],
                            gsem)
                        pltpu.async_copy(
                            ab_hbm.at[idx_d.at[j + 2]], rb.at[(b + 2) % 3],
                            gsem)

                    rab = ra.at[b]
                    rbb = rb.at[b]
                    lgb = lg.at[b]

                    def tk(v, p):
                        return lax.gather(
                            v, p[:, None],
                            dimension_numbers=lax.GatherDimensionNumbers(
                                offset_dims=(), collapsed_slice_dims=(0,),
                                start_index_map=(0,)),
                            slice_sizes=(1,),
                            mode=lax.GatherScatterMode.PROMISE_IN_BOUNDS)

                    def vop(i, cc):
                        g = [jnp.maximum(rab[i, pl.ds(k * 16, 16)]
                                         + rbb[i, pl.ds(H + k * 16, 16)], 0.0)
                             for k in range(H // 16)]
                        s0 = g[0] * w0[0]
                        s1 = g[0] * w1[0]
                        for k in range(1, H // 16):
                            s0 = s0 + g[k] * w0[k]
                            s1 = s1 + g[k] * w1[k]
                        f0 = s0 + tk(s0, px8)
                        f1 = s1 + tk(s1, px8)
                        m = jnp.where(io_lt8, f0, tk(f1, px8))
                        m = m + tk(m, px4)
                        m = m + tk(m, px2)
                        m = m + tk(m, px1)
                        v = tk(m, psel) + bvec
                        plsc.store_scatter(lgb, [io * 0 + i, io], v,
                                           mask=io_lt2)
                        return cc

                    lax.fori_loop(0, C, vop, 0)
                    pltpu.async_copy(
                        lgb, out_hbm.at[pl.ds(wid * EPW + j * C, C)],
                        wsem)
            return carry

        lax.fori_loop(0, (NCH + 2) // 3, pair, 0)
        for _d in range(3):
            pltpu.make_async_copy(
                lg.at[_d], out_hbm.at[pl.ds(0, C)], wsem).wait()

    return pl.kernel(
        body,
        out_type=jax.ShapeDtypeStruct((E, O), _f32),
        mesh=mesh,
        compiler_params=pltpu.CompilerParams(needs_layout_passes=False),
        scratch_types=[
            pltpu.VMEM((NCH, C), jnp.int32),
            pltpu.VMEM((NCH, C), jnp.int32),
            pltpu.VMEM((3, C, 2 * H), _f32),
            pltpu.VMEM((3, C, 2 * H), _f32),
            pltpu.VMEM((3, C, O), _f32),
            pltpu.VMEM((O, H), _f32),
            pltpu.VMEM((16,), _f32),
            pltpu.SemaphoreType.DMA,
            pltpu.SemaphoreType.DMA,
        ],
    )


_seg80 = _seg_kernel(H + 16)
_seg64 = _seg_kernel(H)
_edge = _edge_kernel()

RB = 2000


def _t1_body(x_ref, w1l_ref, w1r_ref, paug_ref, r1_ref):
    xb = x_ref[...]
    p1 = jnp.dot(xb, w1l_ref[...], preferred_element_type=_f32)
    paug_ref[...] = jnp.concatenate(
        [p1, jnp.ones((xb.shape[0], 16), _f32)], axis=1)
    r1_ref[...] = jnp.dot(xb, w1r_ref[...], preferred_element_type=_f32)


_t1 = pl.pallas_call(
    _t1_body,
    grid=(N // RB,),
    in_specs=[
        pl.BlockSpec((RB, D), lambda i: (i, 0)),
        pl.BlockSpec((D, H), lambda i: (0, 0)),
        pl.BlockSpec((D, H), lambda i: (0, 0)),
    ],
    out_specs=[
        pl.BlockSpec((RB, H + 16), lambda i: (i, 0)),
        pl.BlockSpec((RB, H), lambda i: (i, 0)),
    ],
    out_shape=[
        jax.ShapeDtypeStruct((N, H + 16), _f32),
        jax.ShapeDtypeStruct((N, H), _f32),
    ],
)


def _t2_body(tab_ref, r1_ref, b1l_ref, w2l_ref, w2r_ref, p2_ref, r2_ref, inv_ref):
    tab = tab_ref[...]
    agg = tab[0, :, :H] + tab[1, :, :H]
    cnt = tab[0, :, H:H + 1] + tab[1, :, H:H + 1]
    inv = 1.0 / jnp.maximum(cnt, 1.0)
    h1 = jnp.maximum(agg * inv + b1l_ref[...][None, :] + r1_ref[...], 0.0)
    p2_ref[...] = jnp.dot(h1, w2l_ref[...], preferred_element_type=_f32)
    r2_ref[...] = jnp.dot(h1, w2r_ref[...], preferred_element_type=_f32)
    inv_ref[...] = jnp.broadcast_to(inv, (inv.shape[0], 8))


_t2 = pl.pallas_call(
    _t2_body,
    grid=(N // RB,),
    in_specs=[
        pl.BlockSpec((NC, RB, H + 16), lambda i: (0, i, 0)),
        pl.BlockSpec((RB, H), lambda i: (i, 0)),
        pl.BlockSpec((H,), lambda i: (0,)),
        pl.BlockSpec((H, H), lambda i: (0, 0)),
        pl.BlockSpec((H, H), lambda i: (0, 0)),
    ],
    out_specs=[
        pl.BlockSpec((RB, H), lambda i: (i, 0)),
        pl.BlockSpec((RB, H), lambda i: (i, 0)),
        pl.BlockSpec((RB, 8), lambda i: (i, 0)),
    ],
    out_shape=[
        jax.ShapeDtypeStruct((N, H), _f32),
        jax.ShapeDtypeStruct((N, H), _f32),
        jax.ShapeDtypeStruct((N, 8), _f32),
    ],
)


def _t3_body(tab_ref, r2_ref, inv_ref, b2l_ref, wm1_ref, bm1_ref, ab_ref):
    tab = tab_ref[...]
    agg = tab[0] + tab[1]
    inv = inv_ref[...][:, :1]
    h2 = jnp.maximum(agg * inv + b2l_ref[...][None, :] + r2_ref[...], 0.0)
    wm1 = wm1_ref[...]
    a = jnp.dot(h2, wm1[:H], preferred_element_type=_f32) \
        + bm1_ref[...][None, :]
    b = jnp.dot(h2, wm1[H:], preferred_element_type=_f32)
    ab_ref[...] = jnp.concatenate([a, b], axis=1)


_t3 = pl.pallas_call(
    _t3_body,
    grid=(N // RB,),
    in_specs=[
        pl.BlockSpec((NC, RB, H), lambda i: (0, i, 0)),
        pl.BlockSpec((RB, H), lambda i: (i, 0)),
        pl.BlockSpec((RB, 8), lambda i: (i, 0)),
        pl.BlockSpec((H,), lambda i: (0,)),
        pl.BlockSpec((2 * H, H), lambda i: (0, 0)),
        pl.BlockSpec((H,), lambda i: (0,)),
    ],
    out_specs=pl.BlockSpec((RB, 2 * H), lambda i: (i, 0)),
    out_shape=jax.ShapeDtypeStruct((N, 2 * H), _f32),
)


def kernel(x, edge_index, W1l, b1l, W1r, W2l, b2l, W2r, Wm1, bm1, Wm2, bm2):
    src = edge_index[0].reshape(NW, NCH, C)
    dst = edge_index[1].reshape(NW, NCH, C)
    paug, r1 = _t1(x, W1l, W1r)
    tab1 = _seg80(paug, src, dst)
    p2, r2, inv8 = _t2(tab1, r1, b1l, W2l, W2r)
    tab2 = _seg64(p2, src, dst)
    ab = _t3(tab2, r2, inv8, b2l, Wm1, bm1)
    return _edge(ab, src, dst, Wm2.T, jnp.pad(bm2, (0, 14)))


# speedup vs baseline: 1.7686x; 1.1584x over previous
import jax
import jax.numpy as jnp
from jax import lax
from jax.experimental import pallas as pl
from jax.experimental.pallas import tpu as pltpu
from jax.experimental.pallas import tpu_sc as plsc

N = 10000
E = 320000
D = 128
H = 64
O = 2

NC = 2
NS = 16
NW = NC * NS
EPW = E // NW
C = 80
NCH = EPW // C
NP = 10240
RPT = NP // NS
ZB = 128

_f32 = jnp.float32


def _seg_kernel(width):
    mesh = plsc.VectorSubcoreMesh(core_axis_name="c", subcore_axis_name="s")
    gpr = width // 16

    def body(p_hbm, src_hbm, dst_hbm, out_hbm, idx_s, idx_d, rows, zbuf, table,
             isem, gsem, ssem):
        c = lax.axis_index("c")
        s = lax.axis_index("s")
        wid = c * NS + s

        pltpu.async_copy(src_hbm.at[wid], idx_s, isem)
        pltpu.async_copy(dst_hbm.at[wid], idx_d, isem)

        def zs(t, carry):
            zbuf[t // gpr, pl.ds((t % gpr) * 16, 16)] = jnp.zeros((16,), _f32)
            return carry

        lax.fori_loop(0, ZB * gpr, zs, 0)
        base_r = s * RPT
        for z in range(RPT // ZB):
            pltpu.sync_copy(zbuf, table.at[pl.ds(base_r + z * ZB, ZB)])
        pltpu.make_async_copy(src_hbm.at[wid], idx_s, isem).wait()
        pltpu.make_async_copy(dst_hbm.at[wid], idx_d, isem).wait()
        plsc.subcore_barrier()

        pltpu.async_copy(p_hbm.at[idx_s.at[0]], rows.at[0], gsem)
        pltpu.async_copy(p_hbm.at[idx_s.at[1]], rows.at[1], gsem)

        def trip(m, carry):
            for b in range(3):
                j = 3 * m + b

                @pl.when(j < NCH)
                def _():
                    pltpu.make_async_copy(
                        p_hbm.at[pl.ds(0, C)], rows.at[b], gsem).wait()

                    @pl.when(j >= 1)
                    def _():
                        pltpu.make_async_copy(
                            rows.at[b], table.at[pl.ds(0, C)], ssem).wait()

                    @pl.when(j + 2 < NCH)
                    def _():
                        pltpu.async_copy(
                            p_hbm.at[idx_s.at[j + 2]],
                            rows.at[(b + 2) % 3], gsem)

                    pltpu.async_copy(
                        rows.at[b], table.at[idx_d.at[j]], ssem, add=True)
            return carry

        lax.fori_loop(0, (NCH + 2) // 3, trip, 0)
        pltpu.make_async_copy(
            rows.at[0], table.at[pl.ds(0, C)], ssem).wait()
        plsc.subcore_barrier()
        pltpu.sync_copy(table.at[pl.ds(base_r, RPT)],
                        out_hbm.at[c, pl.ds(base_r, RPT)])

    return pl.kernel(
        body,
        out_type=jax.ShapeDtypeStruct((NC, NP, width), _f32),
        mesh=mesh,
        compiler_params=pltpu.CompilerParams(use_tc_tiling_on_sc=False),
        scratch_types=[
            pltpu.VMEM((NCH, C), jnp.int32),
            pltpu.VMEM((NCH, C), jnp.int32),
            pltpu.VMEM((3, C, width), _f32),
            pltpu.VMEM((ZB, width), _f32),
            pltpu.VMEM_SHARED((NP, width), _f32),
            pltpu.SemaphoreType.DMA,
            pltpu.SemaphoreType.DMA,
            pltpu.SemaphoreType.DMA,
        ],
    )


def _edge_kernel():
    mesh = plsc.VectorSubcoreMesh(core_axis_name="c", subcore_axis_name="s")

    def body(ab_hbm, src_hbm, dst_hbm, wm2t_hbm, bm2_hbm, out_hbm,
             idx_s, idx_d, ra, rb, lg, wv, bv, gsem, wsem):
        c = lax.axis_index("c")
        s = lax.axis_index("s")
        wid = c * NS + s
        pltpu.sync_copy(wm2t_hbm, wv)
        pltpu.sync_copy(bm2_hbm, bv)
        pltpu.sync_copy(src_hbm.at[wid], idx_s)
        pltpu.sync_copy(dst_hbm.at[wid], idx_d)

        w0 = [wv[0, pl.ds(k * 16, 16)] for k in range(H // 16)]
        w1 = [wv[1, pl.ds(k * 16, 16)] for k in range(H // 16)]
        io = lax.iota(jnp.int32, 16)
        io_lt8 = io < 8
        px8 = io ^ 8
        px4 = io ^ 4
        px2 = io ^ 2
        px1 = io ^ 1
        lo4 = (io & 4) == 0
        msk4 = (io & 3) == 0
        rowd = (io >> 2) & 1
        cold = io >> 3

        def tk(v, p):
            return lax.gather(
                v, p[:, None],
                dimension_numbers=lax.GatherDimensionNumbers(
                    offset_dims=(), collapsed_slice_dims=(0,),
                    start_index_map=(0,)),
                slice_sizes=(1,),
                mode=lax.GatherScatterMode.PROMISE_IN_BOUNDS)

        bvec = bv[pl.ds(0, 16)]
        bvec2 = tk(bvec, cold)


        pltpu.async_copy(ab_hbm.at[idx_s.at[0]], ra.at[0], gsem)
        pltpu.async_copy(ab_hbm.at[idx_d.at[0]], rb.at[0], gsem)
        pltpu.async_copy(ab_hbm.at[idx_s.at[1]], ra.at[1], gsem)
        pltpu.async_copy(ab_hbm.at[idx_d.at[1]], rb.at[1], gsem)

        def pair(m, carry):
            for b in range(3):
                j = 3 * m + b

                @pl.when(j < NCH)
                def _():
                    pltpu.make_async_copy(
                        ab_hbm.at[pl.ds(0, C)], ra.at[b], gsem).wait()
                    pltpu.make_async_copy(
                        ab_hbm.at[pl.ds(0, C)], rb.at[b], gsem).wait()

                    @pl.when(j >= 3)
                    def _():
                        pltpu.make_async_copy(
                            lg.at[b], out_hbm.at[pl.ds(0, C)],
                            wsem).wait()

                    @pl.when(j + 2 < NCH)
                    def _():
                        pltpu.async_copy(
                            ab_hbm.at[idx_s.at[j + 2]], ra.at[(b + 2) % 3],
                            gsem)
                        pltpu.async_copy(
                            ab_hbm.at[idx_d.at[j + 2]], rb.at[(b + 2) % 3],
                            gsem)

                    rab = ra.at[b]
                    rbb = rb.at[b]
                    lgb = lg.at[b]

                    def vop(i2, cc):
                        acc = []
                        for u in range(2):
                            i = 2 * i2 + u
                            g = [jnp.maximum(
                                     rab[i, pl.ds(k * 16, 16)]
                                     + rbb[i, pl.ds(H + k * 16, 16)], 0.0)
                                 for k in range(H // 16)]
                            s0 = g[0] * w0[0]
                            s1 = g[0] * w1[0]
                            for k in range(1, H // 16):
                                s0 = s0 + g[k] * w0[k]
                                s1 = s1 + g[k] * w1[k]
                            f0 = s0 + tk(s0, px8)
                            f1 = s1 + tk(s1, px8)
                            m = jnp.where(io_lt8, f0, tk(f1, px8))
                            acc.append(m + tk(m, px4))
                        mm = jnp.where(lo4, acc[0], tk(acc[1], px4))
                        mm = mm + tk(mm, px2)
                        mm = mm + tk(mm, px1)
                        v = mm + bvec2
                        plsc.store_scatter(lgb, [2 * i2 + rowd, cold], v,
                                           mask=msk4)
                        return cc
                        return cc

                    lax.fori_loop(0, C // 2, vop, 0)
                    pltpu.async_copy(
                        lgb, out_hbm.at[pl.ds(wid * EPW + j * C, C)],
                        wsem)
            return carry

        lax.fori_loop(0, (NCH + 2) // 3, pair, 0)
        for _d in range(3):
            pltpu.make_async_copy(
                lg.at[_d], out_hbm.at[pl.ds(0, C)], wsem).wait()

    return pl.kernel(
        body,
        out_type=jax.ShapeDtypeStruct((E, O), _f32),
        mesh=mesh,
        compiler_params=pltpu.CompilerParams(needs_layout_passes=False),
        scratch_types=[
            pltpu.VMEM((NCH, C), jnp.int32),
            pltpu.VMEM((NCH, C), jnp.int32),
            pltpu.VMEM((3, C, 2 * H), _f32),
            pltpu.VMEM((3, C, 2 * H), _f32),
            pltpu.VMEM((3, C, O), _f32),
            pltpu.VMEM((O, H), _f32),
            pltpu.VMEM((16,), _f32),
            pltpu.SemaphoreType.DMA,
            pltpu.SemaphoreType.DMA,
        ],
    )


_seg80 = _seg_kernel(H + 16)
_seg64 = _seg_kernel(H)
_edge = _edge_kernel()

RB = 2000


def _t1_body(x_ref, w1l_ref, w1r_ref, paug_ref, r1_ref):
    xb = x_ref[...]
    p1 = jnp.dot(xb, w1l_ref[...], preferred_element_type=_f32)
    paug_ref[...] = jnp.concatenate(
        [p1, jnp.ones((xb.shape[0], 16), _f32)], axis=1)
    r1_ref[...] = jnp.dot(xb, w1r_ref[...], preferred_element_type=_f32)


_t1 = pl.pallas_call(
    _t1_body,
    grid=(N // RB,),
    in_specs=[
        pl.BlockSpec((RB, D), lambda i: (i, 0)),
        pl.BlockSpec((D, H), lambda i: (0, 0)),
        pl.BlockSpec((D, H), lambda i: (0, 0)),
    ],
    out_specs=[
        pl.BlockSpec((RB, H + 16), lambda i: (i, 0)),
        pl.BlockSpec((RB, H), lambda i: (i, 0)),
    ],
    out_shape=[
        jax.ShapeDtypeStruct((N, H + 16), _f32),
        jax.ShapeDtypeStruct((N, H), _f32),
    ],
)


def _t2_body(tab_ref, r1_ref, b1l_ref, w2l_ref, w2r_ref, p2_ref, r2_ref, inv_ref):
    tab = tab_ref[...]
    agg = tab[0, :, :H] + tab[1, :, :H]
    cnt = tab[0, :, H:H + 1] + tab[1, :, H:H + 1]
    inv = 1.0 / jnp.maximum(cnt, 1.0)
    h1 = jnp.maximum(agg * inv + b1l_ref[...][None, :] + r1_ref[...], 0.0)
    p2_ref[...] = jnp.dot(h1, w2l_ref[...], preferred_element_type=_f32)
    r2_ref[...] = jnp.dot(h1, w2r_ref[...], preferred_element_type=_f32)
    inv_ref[...] = jnp.broadcast_to(inv, (inv.shape[0], 8))


_t2 = pl.pallas_call(
    _t2_body,
    grid=(N // RB,),
    in_specs=[
        pl.BlockSpec((NC, RB, H + 16), lambda i: (0, i, 0)),
        pl.BlockSpec((RB, H), lambda i: (i, 0)),
        pl.BlockSpec((H,), lambda i: (0,)),
        pl.BlockSpec((H, H), lambda i: (0, 0)),
        pl.BlockSpec((H, H), lambda i: (0, 0)),
    ],
    out_specs=[
        pl.BlockSpec((RB, H), lambda i: (i, 0)),
        pl.BlockSpec((RB, H), lambda i: (i, 0)),
        pl.BlockSpec((RB, 8), lambda i: (i, 0)),
    ],
    out_shape=[
        jax.ShapeDtypeStruct((N, H), _f32),
        jax.ShapeDtypeStruct((N, H), _f32),
        jax.ShapeDtypeStruct((N, 8), _f32),
    ],
)


def _t3_body(tab_ref, r2_ref, inv_ref, b2l_ref, wm1_ref, bm1_ref, ab_ref):
    tab = tab_ref[...]
    agg = tab[0] + tab[1]
    inv = inv_ref[...][:, :1]
    h2 = jnp.maximum(agg * inv + b2l_ref[...][None, :] + r2_ref[...], 0.0)
    wm1 = wm1_ref[...]
    a = jnp.dot(h2, wm1[:H], preferred_element_type=_f32) \
        + bm1_ref[...][None, :]
    b = jnp.dot(h2, wm1[H:], preferred_element_type=_f32)
    ab_ref[...] = jnp.concatenate([a, b], axis=1)


_t3 = pl.pallas_call(
    _t3_body,
    grid=(N // RB,),
    in_specs=[
        pl.BlockSpec((NC, RB, H), lambda i: (0, i, 0)),
        pl.BlockSpec((RB, H), lambda i: (i, 0)),
        pl.BlockSpec((RB, 8), lambda i: (i, 0)),
        pl.BlockSpec((H,), lambda i: (0,)),
        pl.BlockSpec((2 * H, H), lambda i: (0, 0)),
        pl.BlockSpec((H,), lambda i: (0,)),
    ],
    out_specs=pl.BlockSpec((RB, 2 * H), lambda i: (i, 0)),
    out_shape=jax.ShapeDtypeStruct((N, 2 * H), _f32),
)


def kernel(x, edge_index, W1l, b1l, W1r, W2l, b2l, W2r, Wm1, bm1, Wm2, bm2):
    src = edge_index[0].reshape(NW, NCH, C)
    dst = edge_index[1].reshape(NW, NCH, C)
    paug, r1 = _t1(x, W1l, W1r)
    tab1 = _seg80(paug, src, dst)
    p2, r2, inv8 = _t2(tab1, r1, b1l, W2l, W2r)
    tab2 = _seg64(p2, src, dst)
    ab = _t3(tab2, r2, inv8, b2l, Wm1, bm1)
    return _edge(ab, src, dst, Wm2.T, jnp.pad(bm2, (0, 14)))


# speedup vs baseline: 1.7697x; 1.0007x over previous
import jax
import jax.numpy as jnp
from jax import lax
from jax.experimental import pallas as pl
from jax.experimental.pallas import tpu as pltpu
from jax.experimental.pallas import tpu_sc as plsc

N = 10000
E = 320000
D = 128
H = 64
O = 2

NC = 2
NS = 16
NW = NC * NS
EPW = E // NW
C = 80
NCH = EPW // C
NP = 10240
RPT = NP // NS
ZB = 128

_f32 = jnp.float32


def _seg_kernel(width):
    mesh = plsc.VectorSubcoreMesh(core_axis_name="c", subcore_axis_name="s")
    gpr = width // 16

    def body(p_hbm, src_hbm, dst_hbm, out_hbm, idx_s, idx_d, rows, zbuf, table,
             isem, gsem, ssem):
        c = lax.axis_index("c")
        s = lax.axis_index("s")
        wid = c * NS + s

        pltpu.async_copy(src_hbm.at[wid], idx_s, isem)
        pltpu.async_copy(dst_hbm.at[wid], idx_d, isem)

        def zs(t, carry):
            zbuf[t // gpr, pl.ds((t % gpr) * 16, 16)] = jnp.zeros((16,), _f32)
            return carry

        lax.fori_loop(0, ZB * gpr, zs, 0)
        base_r = s * RPT
        for z in range(RPT // ZB):
            pltpu.sync_copy(zbuf, table.at[pl.ds(base_r + z * ZB, ZB)])
        pltpu.make_async_copy(src_hbm.at[wid], idx_s, isem).wait()
        pltpu.make_async_copy(dst_hbm.at[wid], idx_d, isem).wait()
        plsc.subcore_barrier()

        pltpu.async_copy(p_hbm.at[idx_s.at[0]], rows.at[0], gsem)
        pltpu.async_copy(p_hbm.at[idx_s.at[1]], rows.at[1], gsem)

        def trip(m, carry):
            for b in range(3):
                j = 3 * m + b

                @pl.when(j < NCH)
                def _():
                    pltpu.make_async_copy(
                        p_hbm.at[pl.ds(0, C)], rows.at[b], gsem).wait()

                    @pl.when(j >= 1)
                    def _():
                        pltpu.make_async_copy(
                            rows.at[b], table.at[pl.ds(0, C)], ssem).wait()

                    @pl.when(j + 2 < NCH)
                    def _():
                        pltpu.async_copy(
                            p_hbm.at[idx_s.at[j + 2]],
                            rows.at[(b + 2) % 3], gsem)

                    pltpu.async_copy(
                        rows.at[b], table.at[idx_d.at[j]], ssem, add=True)
            return carry

        lax.fori_loop(0, (NCH + 2) // 3, trip, 0)
        pltpu.make_async_copy(
            rows.at[0], table.at[pl.ds(0, C)], ssem).wait()
        plsc.subcore_barrier()
        pltpu.sync_copy(table.at[pl.ds(base_r, RPT)],
                        out_hbm.at[c, pl.ds(base_r, RPT)])

    return pl.kernel(
        body,
        out_type=jax.ShapeDtypeStruct((NC, NP, width), _f32),
        mesh=mesh,
        compiler_params=pltpu.CompilerParams(use_tc_tiling_on_sc=False),
        scratch_types=[
            pltpu.VMEM((NCH, C), jnp.int32),
            pltpu.VMEM((NCH, C), jnp.int32),
            pltpu.VMEM((3, C, width), _f32),
            pltpu.VMEM((ZB, width), _f32),
            pltpu.VMEM_SHARED((NP, width), _f32),
            pltpu.SemaphoreType.DMA,
            pltpu.SemaphoreType.DMA,
            pltpu.SemaphoreType.DMA,
        ],
    )


def _edge_kernel():
    mesh = plsc.VectorSubcoreMesh(core_axis_name="c", subcore_axis_name="s")

    def body(ab_hbm, src_hbm, dst_hbm, wm2t_hbm, bm2_hbm, out_hbm,
             idx_s, idx_d, ra, rb, lg, wv, bv, gsem, wsem):
        c = lax.axis_index("c")
        s = lax.axis_index("s")
        wid = c * NS + s
        pltpu.sync_copy(wm2t_hbm, wv)
        pltpu.sync_copy(bm2_hbm, bv)
        pltpu.sync_copy(src_hbm.at[wid], idx_s)
        pltpu.sync_copy(dst_hbm.at[wid], idx_d)

        w0 = [wv[0, pl.ds(k * 16, 16)] for k in range(H // 16)]
        w1 = [wv[1, pl.ds(k * 16, 16)] for k in range(H // 16)]
        io = lax.iota(jnp.int32, 16)
        io_lt8 = io < 8
        px8 = io ^ 8
        px4 = io ^ 4
        px2 = io ^ 2
        px1 = io ^ 1
        lo4 = (io & 4) == 0
        msk4 = (io & 3) == 0
        rowd = (io >> 2) & 1
        cold = io >> 3

        def tk(v, p):
            return lax.gather(
                v, p[:, None],
                dimension_numbers=lax.GatherDimensionNumbers(
                    offset_dims=(), collapsed_slice_dims=(0,),
                    start_index_map=(0,)),
                slice_sizes=(1,),
                mode=lax.GatherScatterMode.PROMISE_IN_BOUNDS)

        bvec = bv[pl.ds(0, 16)]
        bvec2 = tk(bvec, cold)

        pltpu.async_copy(ab_hbm.at[idx_s.at[0]], ra.at[0], gsem)
        pltpu.async_copy(ab_hbm.at[idx_d.at[0]], rb.at[0], gsem)
        pltpu.async_copy(ab_hbm.at[idx_s.at[1]], ra.at[1], gsem)
        pltpu.async_copy(ab_hbm.at[idx_d.at[1]], rb.at[1], gsem)

        def pair(m, carry):
            for b in range(3):
                j = 3 * m + b

                @pl.when(j < NCH)
                def _():
                    pltpu.make_async_copy(
                        ab_hbm.at[pl.ds(0, C)], ra.at[b], gsem).wait()
                    pltpu.make_async_copy(
                        ab_hbm.at[pl.ds(0, C)], rb.at[b], gsem).wait()

                    @pl.when(j >= 3)
                    def _():
                        pltpu.make_async_copy(
                            lg.at[b], out_hbm.at[pl.ds(0, C)],
                            wsem).wait()

                    @pl.when(j + 2 < NCH)
                    def _():
                        pltpu.async_copy(
                            ab_hbm.at[idx_s.at[j + 2]], ra.at[(b + 2) % 3],
                            gsem)
                        pltpu.async_copy(
                            ab_hbm.at[idx_d.at[j + 2]], rb.at[(b + 2) % 3],
                            gsem)

                    rab = ra.at[b]
                    rbb = rb.at[b]
                    lgb = lg.at[b]

                    def vop(i2, cc):
                        acc = []
                        for u in range(2):
                            i = 2 * i2 + u
                            g = [jnp.maximum(
                                     rab[i, pl.ds(k * 16, 16)]
                                     + rbb[i, pl.ds(H + k * 16, 16)], 0.0)
                                 for k in range(H // 16)]
                            s0 = g[0] * w0[0]
                            s1 = g[0] * w1[0]
                            for k in range(1, H // 16):
                                s0 = s0 + g[k] * w0[k]
                                s1 = s1 + g[k] * w1[k]
                            f0 = s0 + tk(s0, px8)
                            f1 = s1 + tk(s1, px8)
                            m = jnp.where(io_lt8, f0, tk(f1, px8))
                            acc.append(m + tk(m, px4))
                        mm = jnp.where(lo4, acc[0], tk(acc[1], px4))
                        mm = mm + tk(mm, px2)
                        mm = mm + tk(mm, px1)
                        v = mm + bvec2
                        plsc.store_scatter(lgb, [2 * i2 + rowd, cold], v,
                                           mask=msk4)
                        return cc

                    lax.fori_loop(0, C // 2, vop, 0)
                    pltpu.async_copy(
                        lgb, out_hbm.at[pl.ds(wid * EPW + j * C, C)],
                        wsem)
            return carry

        lax.fori_loop(0, (NCH + 2) // 3, pair, 0)
        for _d in range(3):
            pltpu.make_async_copy(
                lg.at[_d], out_hbm.at[pl.ds(0, C)], wsem).wait()

    return pl.kernel(
        body,
        out_type=jax.ShapeDtypeStruct((E, O), _f32),
        mesh=mesh,
        compiler_params=pltpu.CompilerParams(needs_layout_passes=False),
        scratch_types=[
            pltpu.VMEM((NCH, C), jnp.int32),
            pltpu.VMEM((NCH, C), jnp.int32),
            pltpu.VMEM((3, C, 2 * H), _f32),
            pltpu.VMEM((3, C, 2 * H), _f32),
            pltpu.VMEM((3, C, O), _f32),
            pltpu.VMEM((O, H), _f32),
            pltpu.VMEM((16,), _f32),
            pltpu.SemaphoreType.DMA,
            pltpu.SemaphoreType.DMA,
        ],
    )


_seg80 = _seg_kernel(H + 16)
_seg64 = _seg_kernel(H)
_edge = _edge_kernel()

RB = 2000


def _t1_body(x_ref, w1l_ref, w1r_ref, paug_ref, r1_ref):
    xb = x_ref[...]
    p1 = jnp.dot(xb, w1l_ref[...], preferred_element_type=_f32)
    paug_ref[...] = jnp.concatenate(
        [p1, jnp.ones((xb.shape[0], 16), _f32)], axis=1)
    r1_ref[...] = jnp.dot(xb, w1r_ref[...], preferred_element_type=_f32)


_t1 = pl.pallas_call(
    _t1_body,
    grid=(N // RB,),
    in_specs=[
        pl.BlockSpec((RB, D), lambda i: (i, 0)),
        pl.BlockSpec((D, H), lambda i: (0, 0)),
        pl.BlockSpec((D, H), lambda i: (0, 0)),
    ],
    out_specs=[
        pl.BlockSpec((RB, H + 16), lambda i: (i, 0)),
        pl.BlockSpec((RB, H), lambda i: (i, 0)),
    ],
    out_shape=[
        jax.ShapeDtypeStruct((N, H + 16), _f32),
        jax.ShapeDtypeStruct((N, H), _f32),
    ],
)


def _t2_body(tab_ref, r1_ref, b1l_ref, w2l_ref, w2r_ref, p2_ref, r2_ref, inv_ref):
    tab = tab_ref[...]
    agg = tab[0, :, :H] + tab[1, :, :H]
    cnt = tab[0, :, H:H + 1] + tab[1, :, H:H + 1]
    inv = 1.0 / jnp.maximum(cnt, 1.0)
    h1 = jnp.maximum(agg * inv + b1l_ref[...][None, :] + r1_ref[...], 0.0)
    p2_ref[...] = jnp.dot(h1, w2l_ref[...], preferred_element_type=_f32)
    r2_ref[...] = jnp.dot(h1, w2r_ref[...], preferred_element_type=_f32)
    inv_ref[...] = jnp.broadcast_to(inv, (inv.shape[0], 8))


_t2 = pl.pallas_call(
    _t2_body,
    grid=(N // RB,),
    in_specs=[
        pl.BlockSpec((NC, RB, H + 16), lambda i: (0, i, 0)),
        pl.BlockSpec((RB, H), lambda i: (i, 0)),
        pl.BlockSpec((H,), lambda i: (0,)),
        pl.BlockSpec((H, H), lambda i: (0, 0)),
        pl.BlockSpec((H, H), lambda i: (0, 0)),
    ],
    out_specs=[
        pl.BlockSpec((RB, H), lambda i: (i, 0)),
        pl.BlockSpec((RB, H), lambda i: (i, 0)),
        pl.BlockSpec((RB, 8), lambda i: (i, 0)),
    ],
    out_shape=[
        jax.ShapeDtypeStruct((N, H), _f32),
        jax.ShapeDtypeStruct((N, H), _f32),
        jax.ShapeDtypeStruct((N, 8), _f32),
    ],
)


def _t3_body(tab_ref, r2_ref, inv_ref, b2l_ref, wm1_ref, bm1_ref, ab_ref):
    tab = tab_ref[...]
    agg = tab[0] + tab[1]
    inv = inv_ref[...][:, :1]
    h2 = jnp.maximum(agg * inv + b2l_ref[...][None, :] + r2_ref[...], 0.0)
    wm1 = wm1_ref[...]
    a = jnp.dot(h2, wm1[:H], preferred_element_type=_f32) \
        + bm1_ref[...][None, :]
    b = jnp.dot(h2, wm1[H:], preferred_element_type=_f32)
    ab_ref[...] = jnp.concatenate([a, b], axis=1)


_t3 = pl.pallas_call(
    _t3_body,
    grid=(N // RB,),
    in_specs=[
        pl.BlockSpec((NC, RB, H), lambda i: (0, i, 0)),
        pl.BlockSpec((RB, H), lambda i: (i, 0)),
        pl.BlockSpec((RB, 8), lambda i: (i, 0)),
        pl.BlockSpec((H,), lambda i: (0,)),
        pl.BlockSpec((2 * H, H), lambda i: (0, 0)),
        pl.BlockSpec((H,), lambda i: (0,)),
    ],
    out_specs=pl.BlockSpec((RB, 2 * H), lambda i: (i, 0)),
    out_shape=jax.ShapeDtypeStruct((N, 2 * H), _f32),
)


def kernel(x, edge_index, W1l, b1l, W1r, W2l, b2l, W2r, Wm1, bm1, Wm2, bm2):
    src = edge_index[0].reshape(NW, NCH, C)
    dst = edge_index[1].reshape(NW, NCH, C)
    paug, r1 = _t1(x, W1l, W1r)
    tab1 = _seg80(paug, src, dst)
    p2, r2, inv8 = _t2(tab1, r1, b1l, W2l, W2r)
    tab2 = _seg64(p2, src, dst)
    ab = _t3(tab2, r2, inv8, b2l, Wm1, bm1)
    return _edge(ab, src, dst, Wm2.T, jnp.pad(bm2, (0, 14)))
